# Initial kernel scaffold; baseline (speedup 1.0000x reference)
import jax
import jax.numpy as jnp
from jax.experimental import pallas as pl


def kernel(x, embedding_table):
    raise NotImplementedError("write your pallas kernel here")


# speedup vs baseline: 1.0939x; 1.0939x over previous
import functools

import jax
import jax.numpy as jnp
from jax import lax
from jax.experimental import pallas as pl
from jax.experimental.pallas import tpu as pltpu
from jax.experimental.pallas import tpu_sc as plsc

VOCAB = 1000000
EMBED = 32
B_TOTAL = 16384 * 50

_info = plsc.get_sparse_core_info()
_NC, _NS = _info.num_cores, _info.num_subcores
_NW = _NC * _NS
_B_PER_W = B_TOTAL // _NW
_CHUNK = 1024
_NCHUNK = _B_PER_W // _CHUNK


def _make_gather():
  mesh = plsc.VectorSubcoreMesh(core_axis_name="c", subcore_axis_name="s")

  @functools.partial(
      pl.kernel,
      mesh=mesh,
      out_type=jax.ShapeDtypeStruct((B_TOTAL, EMBED), jnp.float32),
      scratch_types=[
          pltpu.VMEM((_CHUNK,), jnp.int32),
          pltpu.VMEM((_CHUNK, EMBED), jnp.float32),
          pltpu.SemaphoreType.DMA,
      ],
      compiler_params=pltpu.CompilerParams(use_tc_tiling_on_sc=False),
  )
  def gather_k(table_hbm, idx_hbm, out_hbm, idx_v, rows_v, sem):
    wid = lax.axis_index("s") * _NC + lax.axis_index("c")
    base = wid * _B_PER_W
    for c in range(_NCHUNK):
      off = base + c * _CHUNK
      pltpu.sync_copy(idx_hbm.at[pl.ds(off, _CHUNK)], idx_v)
      pltpu.async_copy(table_hbm.at[idx_v], rows_v, sem).wait()
      pltpu.sync_copy(rows_v, out_hbm.at[pl.ds(off, _CHUNK)])

  return gather_k


_gather = _make_gather()


@jax.jit
def kernel(x, embedding_table):
  idx = x.reshape(-1).astype(jnp.int32)
  out = _gather(embedding_table, idx)
  return out.reshape(x.shape[0], x.shape[1], EMBED)


# speedup vs baseline: 1.1125x; 1.0170x over previous
import functools

import jax
import jax.numpy as jnp
from jax import lax
from jax.experimental import pallas as pl
from jax.experimental.pallas import tpu as pltpu
from jax.experimental.pallas import tpu_sc as plsc

VOCAB = 1000000
EMBED = 32
B_TOTAL = 16384 * 50

_info = plsc.get_sparse_core_info()
_NC, _NS = _info.num_cores, _info.num_subcores
_NW = _NC * _NS
_B_PER_W = B_TOTAL // _NW
_CHUNK = 1024
_NCHUNK = _B_PER_W // _CHUNK
_NBUF = 3


def _make_gather():
  mesh = plsc.VectorSubcoreMesh(core_axis_name="c", subcore_axis_name="s")

  @functools.partial(
      pl.kernel,
      mesh=mesh,
      out_type=jax.ShapeDtypeStruct((B_TOTAL, EMBED), jnp.float32),
      scratch_types=[
          pltpu.VMEM((_B_PER_W,), jnp.int32),
          pltpu.VMEM((_NBUF, _CHUNK, EMBED), jnp.float32),
          pltpu.SemaphoreType.DMA,
          pltpu.SemaphoreType.DMA,
          pltpu.SemaphoreType.DMA,
          pltpu.SemaphoreType.DMA,
          pltpu.SemaphoreType.DMA,
          pltpu.SemaphoreType.DMA,
      ],
      compiler_params=pltpu.CompilerParams(use_tc_tiling_on_sc=False),
  )
  def gather_k(table_hbm, idx_hbm, out_hbm, idx_v, rows_v,
               g0, g1, g2, s0, s1, s2):
    gsem = [g0, g1, g2]
    ssem = [s0, s1, s2]
    wid = lax.axis_index("s") * _NC + lax.axis_index("c")
    base = wid * _B_PER_W
    pltpu.sync_copy(idx_hbm.at[pl.ds(base, _B_PER_W)], idx_v)

    gd = [None] * _NCHUNK
    sd = [None] * _NCHUNK

    def start_store(c):
      b = c % _NBUF
      sd[c] = pltpu.async_copy(
          rows_v.at[b], out_hbm.at[pl.ds(base + c * _CHUNK, _CHUNK)],
          ssem[b])

    for c in range(_NCHUNK):
      b = c % _NBUF
      if c >= _NBUF:
        sd[c - _NBUF].wait()
      gd[c] = pltpu.async_copy(
          table_hbm.at[idx_v.at[pl.ds(c * _CHUNK, _CHUNK)]], rows_v.at[b],
          gsem[b])
      if c >= 1:
        gd[c - 1].wait()
        start_store(c - 1)

    gd[_NCHUNK - 1].wait()
    start_store(_NCHUNK - 1)
    for c in range(max(0, _NCHUNK - _NBUF), _NCHUNK):
      sd[c].wait()

  return gather_k


_gather = _make_gather()


@jax.jit
def kernel(x, embedding_table):
  idx = x.reshape(-1).astype(jnp.int32)
  out = _gather(embedding_table, idx)
  return out.reshape(x.shape[0], x.shape[1], EMBED)


# speedup vs baseline: 1.1136x; 1.0010x over previous
import functools

import jax
import jax.numpy as jnp
from jax import lax
from jax.experimental import pallas as pl
from jax.experimental.pallas import tpu as pltpu
from jax.experimental.pallas import tpu_sc as plsc

VOCAB = 1000000
EMBED = 32
B_TOTAL = 16384 * 50

_info = plsc.get_sparse_core_info()
_NC, _NS = _info.num_cores, _info.num_subcores
_NW = _NC * _NS
_B_PER_W = B_TOTAL // _NW
_CHUNK = 800
_NCHUNK = _B_PER_W // _CHUNK
_NBUF = 4
_GDEPTH = 2


def _make_gather():
  mesh = plsc.VectorSubcoreMesh(core_axis_name="c", subcore_axis_name="s")

  @functools.partial(
      pl.kernel,
      mesh=mesh,
      out_type=jax.ShapeDtypeStruct((B_TOTAL, EMBED), jnp.float32),
      scratch_types=[
          pltpu.VMEM((_B_PER_W,), jnp.int32),
          pltpu.VMEM((_NBUF, _CHUNK, EMBED), jnp.float32),
      ] + [pltpu.SemaphoreType.DMA] * (2 * _NBUF),
      compiler_params=pltpu.CompilerParams(use_tc_tiling_on_sc=False),
  )
  def gather_k(table_hbm, idx_hbm, out_hbm, idx_v, rows_v, *sems):
    gsem = list(sems[:_NBUF])
    ssem = list(sems[_NBUF:])
    wid = lax.axis_index("s") * _NC + lax.axis_index("c")
    base = wid * _B_PER_W
    pltpu.sync_copy(idx_hbm.at[pl.ds(base, _B_PER_W)], idx_v)

    gd = [None] * _NCHUNK
    sd = [None] * _NCHUNK

    def start_store(c):
      b = c % _NBUF
      sd[c] = pltpu.async_copy(
          rows_v.at[b], out_hbm.at[pl.ds(base + c * _CHUNK, _CHUNK)],
          ssem[b])

    for c in range(_NCHUNK):
      b = c % _NBUF
      if c >= _NBUF:
        sd[c - _NBUF].wait()
      gd[c] = pltpu.async_copy(
          table_hbm.at[idx_v.at[pl.ds(c * _CHUNK, _CHUNK)]], rows_v.at[b],
          gsem[b])
      if c >= _GDEPTH:
        gd[c - _GDEPTH].wait()
        start_store(c - _GDEPTH)

    for c in range(_NCHUNK - _GDEPTH, _NCHUNK):
      gd[c].wait()
      start_store(c)
    for c in range(max(0, _NCHUNK - _NBUF), _NCHUNK):
      sd[c].wait()

  return gather_k


_gather = _make_gather()


@jax.jit
def kernel(x, embedding_table):
  idx = x.reshape(-1).astype(jnp.int32)
  out = _gather(embedding_table, idx)
  return out.reshape(x.shape[0], x.shape[1], EMBED)
